# Initial kernel scaffold; baseline (speedup 1.0000x reference)
import jax
import jax.numpy as jnp
from jax.experimental import pallas as pl


def kernel(X, edge_index, edge_weight, skip, H, C, params):
    raise NotImplementedError("write your pallas kernel here")


# speedup vs baseline: 5.2337x; 5.2337x over previous
import functools
import numpy as np
import jax
import jax.numpy as jnp
from jax import lax
from jax.experimental import pallas as pl
from jax.experimental.pallas import tpu as pltpu
from jax.experimental.pallas import tpu_sc as plsc

N = 10000
E = 320000
D = 128
SK = 2
NP = 10240
KVW = 256
CH = 80
NR1 = 10112
HN = NP // 2
EPT1 = E // 16
EPT5 = E // 32
CH5 = 2000

_mesh = plsc.VectorSubcoreMesh(core_axis_name="c", subcore_axis_name="s")
_sc_params = pltpu.CompilerParams(needs_layout_passes=False)


def _zero_vmem(ref, rows, vregs):
    def body(r, _):
        for j in range(vregs):
            ref[r, pl.ds(j * 16, 16)] = jnp.zeros((16,), jnp.float32)
        return 0
    lax.fori_loop(0, rows, body, 0)


NR1H = 5120


@functools.partial(
    pl.kernel, mesh=_mesh, compiler_params=_sc_params,
    out_type=jax.ShapeDtypeStruct((2, NP, D), jnp.float32),
    scratch_types=[
        pltpu.VMEM((CH,), jnp.int32),
        pltpu.VMEM((CH,), jnp.int32),
        pltpu.VMEM((CH,), jnp.int32),
        pltpu.VMEM((CH,), jnp.float32),
        pltpu.VMEM((CH, D), jnp.float32),
        pltpu.VMEM((128, D), jnp.float32),
        pltpu.VMEM_SHARED((NR1H, D), jnp.float32),
        pltpu.SemaphoreType.DMA,
    ],
)
def _k1_msg(table_hbm, src_hbm, dst_hbm, ew_hbm, msg_hbm,
            src_v, dst_v, dloc_v, ew_v, rows_v, buf_v, acc_sh, sem):
    cid = lax.axis_index("c")
    sid = lax.axis_index("s")
    base0 = sid * EPT1
    lo1 = cid * NR1H

    for tbl in range(2):
        _zero_vmem(buf_v, 128, D // 16)
        for off_r, n_r in ((0, 128), (128, 128), (256, 64)):
            pltpu.sync_copy(buf_v.at[pl.ds(0, n_r)],
                            acc_sh.at[pl.ds(sid * 320 + off_r, n_r)])
        plsc.subcore_barrier()

        def chunk(ci, _):
            eb = base0 + ci * CH
            pltpu.sync_copy(src_hbm.at[pl.ds(eb, CH)], src_v)
            pltpu.sync_copy(dst_hbm.at[pl.ds(eb, CH)], dst_v)
            pltpu.sync_copy(ew_hbm.at[pl.ds(eb, CH)], ew_v)
            for g in range(CH // 16):
                sg = src_v[pl.ds(g * 16, 16)]
                src_v[pl.ds(g * 16, 16)] = sg + tbl * NP
                dg = dst_v[pl.ds(g * 16, 16)] - lo1
                rm = jnp.where((dg >= 0) & (dg < NR1H), 1.0, 0.0)
                dloc_v[pl.ds(g * 16, 16)] = jnp.minimum(
                    jnp.maximum(dg, 0), NR1H - 1)
                ew_v[pl.ds(g * 16, 16)] = ew_v[pl.ds(g * 16, 16)] * rm
            pltpu.async_copy(table_hbm.at[src_v], rows_v, sem).wait()

            def group(gi, _):
                wv = ew_v[pl.ds(gi * 16, 16)]
                for e16 in range(16):
                    w = wv[e16]
                    r = gi * 16 + e16
                    for j in range(D // 16):
                        rows_v[r, pl.ds(j * 16, 16)] = \
                            rows_v[r, pl.ds(j * 16, 16)] * w
                return 0
            lax.fori_loop(0, CH // 16, group, 0)
            pltpu.sync_copy(rows_v, acc_sh.at[dloc_v], add=True)
            return 0

        lax.fori_loop(0, EPT1 // CH, chunk, 0)
        plsc.subcore_barrier()
        for off_r, n_r in ((0, 128), (128, 128), (256, 64)):
            rb = sid * 320 + off_r
            pltpu.sync_copy(acc_sh.at[pl.ds(rb, n_r)], buf_v.at[pl.ds(0, n_r)])
            pltpu.sync_copy(buf_v.at[pl.ds(0, n_r)],
                            msg_hbm.at[tbl, pl.ds(lo1 + rb, n_r)])
        plsc.subcore_barrier()


@functools.partial(
    pl.kernel, mesh=_mesh, compiler_params=_sc_params,
    out_type=[jax.ShapeDtypeStruct((NP, D), jnp.float32),
              jax.ShapeDtypeStruct((3 * NP,), jnp.float32),
              jax.ShapeDtypeStruct((96 * HN,), jnp.float32),
              jax.ShapeDtypeStruct((E,), jnp.float32)],
    scratch_types=[
        pltpu.VMEM((CH,), jnp.int32),
        pltpu.VMEM((CH,), jnp.int32),
        pltpu.VMEM((CH,), jnp.int32),
        pltpu.VMEM((CH,), jnp.float32),
        pltpu.VMEM((CH,), jnp.float32),
        pltpu.VMEM((CH,), jnp.float32),
        pltpu.VMEM((CH, D), jnp.float32),
        pltpu.VMEM((CH, KVW), jnp.float32),
        pltpu.VMEM((CH, D), jnp.float32),
        pltpu.VMEM((64, D), jnp.float32),
        pltpu.VMEM((16,), jnp.float32),
        pltpu.VMEM((HN,), jnp.float32),
        pltpu.VMEM((HN,), jnp.float32),
        pltpu.VMEM((HN,), jnp.float32),
        pltpu.VMEM((HN,), jnp.float32),
        pltpu.VMEM((HN,), jnp.float32),
        pltpu.VMEM((320,), jnp.float32),
        pltpu.VMEM((320,), jnp.float32),
        pltpu.VMEM_SHARED((HN, D), jnp.float32),
        pltpu.SemaphoreType.DMA,
    ],
)
def _k3_tconv1(q_hbm, kv_hbm, qe0_hbm, qe1_hbm, src_hbm, dst_hbm,
               ew0_hbm, ew1_hbm, w2_hbm,
               acc_hbm, dn3_hbm, part_hbm, e2_hbm,
               src_v, dst_v, dloc_v, ew0_v, ew1_v, e2_v, qrows, kvrows, srows,
               buf_v, w2_v, qe0_v, qe1_v, den_t, dw0_t, dw1_t, tmp_v, red_v,
               acc_sh, sem):
    cid = lax.axis_index("c")
    sid = lax.axis_index("s")
    lo = cid * HN
    _zero_vmem(buf_v, 64, D // 16)
    for zc in range(5):
        pltpu.sync_copy(buf_v, acc_sh.at[pl.ds(sid * 320 + zc * 64, 64)])
    pltpu.sync_copy(w2_hbm, w2_v)
    pltpu.sync_copy(qe0_hbm.at[pl.ds(lo, HN)], qe0_v)
    pltpu.sync_copy(qe1_hbm.at[pl.ds(lo, HN)], qe1_v)

    def zs(r, _):
        den_t[pl.ds(r * 16, 16)] = jnp.zeros((16,), jnp.float32)
        dw0_t[pl.ds(r * 16, 16)] = jnp.zeros((16,), jnp.float32)
        dw1_t[pl.ds(r * 16, 16)] = jnp.zeros((16,), jnp.float32)
        return 0
    lax.fori_loop(0, HN // 16, zs, 0)
    plsc.subcore_barrier()

    wid = sid * 2 + cid
    base0 = sid * EPT1
    w2g = w2_v[...]
    w20 = w2g[0]
    w21 = w2g[1]
    lanes = lax.iota(jnp.int32, 16)
    z16 = jnp.zeros((16,), jnp.int32)
    perms = [lanes ^ (1 << k) for k in range(4)]

    dnums = lax.GatherDimensionNumbers(
        offset_dims=(), collapsed_slice_dims=(0,), start_index_map=(0,))

    def _lanesum(d):
        for p_ in perms:
            d = d + lax.gather(
                d, p_[:, None], dnums, slice_sizes=(1,),
                mode=lax.GatherScatterMode.PROMISE_IN_BOUNDS,
                unique_indices=True, indices_are_sorted=False)
        return d

    def chunk(ci, _):
        eb = base0 + ci * CH
        pltpu.sync_copy(src_hbm.at[pl.ds(eb, CH)], src_v)
        pltpu.sync_copy(dst_hbm.at[pl.ds(eb, CH)], dst_v)
        pltpu.sync_copy(ew0_hbm.at[pl.ds(eb, CH)], ew0_v)
        pltpu.sync_copy(ew1_hbm.at[pl.ds(eb, CH)], ew1_v)
        cp1 = pltpu.async_copy(q_hbm.at[dst_v], qrows, sem)
        cp2 = pltpu.async_copy(kv_hbm.at[src_v], kvrows, sem)
        cp1.wait()
        cp2.wait()

        def group(gi, _):
            ewv0 = ew0_v[pl.ds(gi * 16, 16)]
            ewv1 = ew1_v[pl.ds(gi * 16, 16)]
            e2_v[pl.ds(gi * 16, 16)] = ewv0 * w20 + ewv1 * w21
            di16 = dst_v[pl.ds(gi * 16, 16)]
            dil = di16 - lo
            rm = jnp.where((dil >= 0) & (dil < HN), 1.0, 0.0).astype(jnp.float32)
            dil = jnp.minimum(jnp.maximum(dil, 0), HN - 1)
            dloc_v[pl.ds(gi * 16, 16)] = dil
            qe0g = plsc.load_gather(qe0_v, [dil])
            qe1g = plsc.load_gather(qe1_v, [dil])
            aext = qe0g * ewv0 + qe1g * ewv1
            evec = jnp.zeros((16,), jnp.float32)
            for e16 in range(16):
                r = gi * 16 + e16
                d = qrows[r, pl.ds(0, 16)] * kvrows[r, pl.ds(0, 16)]
                for j in range(1, D // 16):
                    d = d + qrows[r, pl.ds(j * 16, 16)] * kvrows[r, pl.ds(j * 16, 16)]
                sv = _lanesum(d)
                exm = jnp.exp(sv + aext[e16]) * rm[e16]
                for j in range(D // 16):
                    srows[r, pl.ds(j * 16, 16)] = kvrows[r, pl.ds(D + j * 16, 16)] * exm
                evec = jnp.where(lanes == e16, exm, evec)
            plsc.addupdate_scatter(den_t, [dil], evec)
            plsc.addupdate_scatter(dw0_t, [dil], evec * ewv0)
            plsc.addupdate_scatter(dw1_t, [dil], evec * ewv1)
            return 0
        lax.fori_loop(0, CH // 16, group, 0)

        @pl.when(cid == 0)
        def _():
            pltpu.sync_copy(e2_v, e2_hbm.at[pl.ds(eb, CH)])
        pltpu.sync_copy(srows, acc_sh.at[dloc_v], add=True)
        return 0

    lax.fori_loop(0, EPT1 // CH, chunk, 0)

    pltpu.sync_copy(den_t, part_hbm.at[pl.ds((wid * 3) * HN, HN)])
    pltpu.sync_copy(dw0_t, part_hbm.at[pl.ds((wid * 3 + 1) * HN, HN)])
    pltpu.sync_copy(dw1_t, part_hbm.at[pl.ds((wid * 3 + 2) * HN, HN)])
    plsc.subcore_barrier()
    for zc in range(5):
        rb = sid * 320 + zc * 64
        pltpu.sync_copy(acc_sh.at[pl.ds(rb, 64)], buf_v)
        pltpu.sync_copy(buf_v, acc_hbm.at[pl.ds(lo + rb, 64)])
    rb = sid * 320
    for half in range(3):
        def rz(g, _):
            red_v[pl.ds(g * 16, 16)] = jnp.zeros((16,), jnp.float32)
            return 0
        lax.fori_loop(0, 320 // 16, rz, 0)

        def racc(t, _):
            pltpu.sync_copy(
                part_hbm.at[pl.ds(((t * 2 + cid) * 3 + half) * HN + rb, 320)],
                tmp_v)

            def radd(g, _):
                red_v[pl.ds(g * 16, 16)] = (red_v[pl.ds(g * 16, 16)]
                                            + tmp_v[pl.ds(g * 16, 16)])
                return 0
            lax.fori_loop(0, 320 // 16, radd, 0)
            return 0
        lax.fori_loop(0, 16, racc, 0)
        pltpu.sync_copy(red_v,
                        dn3_hbm.at[pl.ds(half * NP + lo + rb, 320)])


@functools.partial(
    pl.kernel, mesh=_mesh, compiler_params=_sc_params,
    out_type=jax.ShapeDtypeStruct((64 * NP,), jnp.float32),
    scratch_types=[
        pltpu.VMEM((NP,), jnp.float32),
        pltpu.VMEM((NP,), jnp.float32),
        pltpu.VMEM((NP,), jnp.float32),
        pltpu.VMEM((NP,), jnp.float32),
        pltpu.VMEM((NP,), jnp.float32),
        pltpu.VMEM((CH5,), jnp.int32),
        pltpu.VMEM((CH5,), jnp.int32),
        pltpu.VMEM((CH5,), jnp.float32),
        pltpu.SemaphoreType.DMA,
    ],
)
def _k5_tconv2(q2_hbm, k2_hbm, v2_hbm, src_hbm, dst_hbm, e2_hbm, dn_hbm,
               q2_v, k2_v, v2_v, den_t, num_t, src_v, dst_v, e2_v, sem):
    cid = lax.axis_index("c")
    sid = lax.axis_index("s")
    pltpu.sync_copy(q2_hbm, q2_v)
    pltpu.sync_copy(k2_hbm, k2_v)
    pltpu.sync_copy(v2_hbm, v2_v)

    def zbody(r, _):
        den_t[pl.ds(r * 16, 16)] = jnp.zeros((16,), jnp.float32)
        num_t[pl.ds(r * 16, 16)] = jnp.zeros((16,), jnp.float32)
        return 0
    lax.fori_loop(0, NP // 16, zbody, 0)

    wid = sid * 2 + cid
    base0 = wid * EPT5
    z16 = jnp.zeros((16,), jnp.int32)

    def chunk(ci, _):
        eb = base0 + ci * CH5
        pltpu.sync_copy(src_hbm.at[pl.ds(eb, CH5)], src_v)
        pltpu.sync_copy(dst_hbm.at[pl.ds(eb, CH5)], dst_v)
        pltpu.sync_copy(e2_hbm.at[pl.ds(eb, CH5)], e2_v)

        def group(g, _):
            si = src_v[pl.ds(g * 16, 16)]
            di = dst_v[pl.ds(g * 16, 16)]
            e2g = e2_v[pl.ds(g * 16, 16)]
            qv = plsc.load_gather(q2_v, [di])
            kv = plsc.load_gather(k2_v, [si])
            vv = plsc.load_gather(v2_v, [si])
            ex = jnp.exp(qv * (kv + e2g))
            plsc.addupdate_scatter(den_t, [di], ex)
            plsc.addupdate_scatter(num_t, [di], ex * (vv + e2g))
            return 0
        lax.fori_loop(0, CH5 // 16, group, 0)
        return 0
    lax.fori_loop(0, EPT5 // CH5, chunk, 0)

    pltpu.sync_copy(den_t, dn_hbm.at[pl.ds((wid * 2) * NP, NP)])
    pltpu.sync_copy(num_t, dn_hbm.at[pl.ds((wid * 2 + 1) * NP, NP)])


def _ln_rows(x, g, b):
    m = jnp.mean(x, axis=-1, keepdims=True)
    v = jnp.mean((x - m) * (x - m), axis=-1, keepdims=True)
    return (x - m) / jnp.sqrt(v + 1e-5) * g + b


def _k2_body(x_ref, mx_ref, h_ref, mh_ref, c_ref, sk_ref,
             wx_ref, wh_ref, bcat_ref, wp_ref, lng_ref, lnb_ref,
             w1_ref, we1_ref,
             hid_ref, cel_ref, q_ref, qe2_ref, kv_ref, r_ref):
    a = x_ref[:] + mx_ref[:]
    bh = h_ref[:] + mh_ref[:]
    c = c_ref[:]
    z = (jnp.dot(a, wx_ref[:], preferred_element_type=jnp.float32)
         + jnp.dot(bh, wh_ref[:], preferred_element_type=jnp.float32)
         + bcat_ref[:])
    zi = z[:, 0:D]
    zf = z[:, D:2 * D]
    zg = z[:, 2 * D:3 * D]
    zo = z[:, 3 * D:4 * D]
    wpi = wp_ref[0:1, :]
    wpf = wp_ref[1:2, :]
    wpo = wp_ref[2:3, :]
    ig = jax.nn.sigmoid(zi + wpi * c)
    fg = jax.nn.sigmoid(zf + wpf * c)
    gg = jnp.tanh(zg)
    cn = fg * c + ig * gg
    og = jax.nn.sigmoid(zo + wpo * cn)
    hn = og * jnp.tanh(cn)
    hid_ref[:] = _ln_rows(hn, lng_ref[1:2, :], lnb_ref[1:2, :])
    cel_ref[:] = _ln_rows(cn, lng_ref[2:3, :], lnb_ref[2:3, :])
    o0 = jax.nn.relu(_ln_rows(hn, lng_ref[0:1, :], lnb_ref[0:1, :]))
    x130 = jnp.concatenate([o0, sk_ref[:]], axis=1)
    qkvr = jnp.dot(x130, w1_ref[:], preferred_element_type=jnp.float32)
    inv = np.float32(1.0 / np.sqrt(float(D)))
    q = qkvr[:, 0:D] * inv
    r_ref[:] = qkvr[:, 3 * D:4 * D]
    q_ref[:] = q
    qe2_ref[:] = jnp.dot(q, we1_ref[:].T, preferred_element_type=jnp.float32)
    kv_ref[:] = qkvr[:, D:3 * D]


def _k2_call(xp, mx, hp, mh, cp, skp, wx, wh, bcat, wp, lng, lnb, w1, we1):
    BLK = 512
    grid = (NP // BLK,)
    row = lambda i: (i, 0)
    full = lambda i: (0, 0)
    return pl.pallas_call(
        _k2_body,
        grid=grid,
        in_specs=[
            pl.BlockSpec((BLK, D), row), pl.BlockSpec((BLK, D), row),
            pl.BlockSpec((BLK, D), row), pl.BlockSpec((BLK, D), row),
            pl.BlockSpec((BLK, D), row), pl.BlockSpec((BLK, SK), row),
            pl.BlockSpec((D, 4 * D), full), pl.BlockSpec((D, 4 * D), full),
            pl.BlockSpec((1, 4 * D), full), pl.BlockSpec((3, D), full),
            pl.BlockSpec((3, D), full), pl.BlockSpec((3, D), full),
            pl.BlockSpec((D + SK, 4 * D), full), pl.BlockSpec((2, D), full),
        ],
        out_specs=[
            pl.BlockSpec((BLK, D), row), pl.BlockSpec((BLK, D), row),
            pl.BlockSpec((BLK, D), row), pl.BlockSpec((BLK, 2), row),
            pl.BlockSpec((BLK, KVW), row), pl.BlockSpec((BLK, D), row),
        ],
        out_shape=[
            jax.ShapeDtypeStruct((NP, D), jnp.float32),
            jax.ShapeDtypeStruct((NP, D), jnp.float32),
            jax.ShapeDtypeStruct((NP, D), jnp.float32),
            jax.ShapeDtypeStruct((NP, 2), jnp.float32),
            jax.ShapeDtypeStruct((NP, KVW), jnp.float32),
            jax.ShapeDtypeStruct((NP, D), jnp.float32),
        ],
    )(xp, mx, hp, mh, cp, skp, wx, wh, bcat, wp, lng, lnb, w1, we1)


def _k4_body(acc_ref, dst_ref, r_ref, we1_ref, b1_ref, w2_ref, qkvr2_ref):
    numv = acc_ref[:]
    st = dst_ref[:]
    den = st[:, 0:1]
    dew = st[:, 1:3]
    agg = (numv + jnp.dot(dew, we1_ref[:], preferred_element_type=jnp.float32)) \
        / (den + 1e-16)
    out1 = jax.nn.relu(agg + r_ref[:] + b1_ref[:])
    qkvr2_ref[:] = jnp.dot(out1, w2_ref[:], preferred_element_type=jnp.float32)


def _k4_call(acc, dstats_t, r1, we1, b1, w2):
    BLK = 512
    grid = (NP // BLK,)
    return pl.pallas_call(
        _k4_body,
        grid=grid,
        in_specs=[
            pl.BlockSpec((BLK, D), lambda i: (i, 0)),
            pl.BlockSpec((BLK, 3), lambda i: (i, 0)),
            pl.BlockSpec((BLK, D), lambda i: (i, 0)),
            pl.BlockSpec((2, D), lambda i: (0, 0)),
            pl.BlockSpec((1, D), lambda i: (0, 0)),
            pl.BlockSpec((D, 4), lambda i: (0, 0)),
        ],
        out_specs=pl.BlockSpec((BLK, 4), lambda i: (i, 0)),
        out_shape=jax.ShapeDtypeStruct((NP, 4), jnp.float32),
    )(acc, dstats_t, r1, we1, b1, w2)


def _k6_body(dn_ref, r2_ref, b2_ref, out_ref):
    den = dn_ref[0:1, :]
    num = dn_ref[1:2, :]
    for t in range(1, 32):
        den = den + dn_ref[2 * t:2 * t + 1, :]
        num = num + dn_ref[2 * t + 1:2 * t + 2, :]
    out_ref[:] = jax.nn.sigmoid(num / (den + 1e-16) + r2_ref[:] + b2_ref[0, 0])


def _k6_call(dn, r2, b2):
    return pl.pallas_call(
        _k6_body,
        in_specs=[
            pl.BlockSpec((64, NP), lambda: (0, 0)),
            pl.BlockSpec((1, NP), lambda: (0, 0)),
            pl.BlockSpec((1, 1), lambda: (0, 0)),
        ],
        out_specs=pl.BlockSpec((1, NP), lambda: (0, 0)),
        out_shape=jax.ShapeDtypeStruct((1, NP), jnp.float32),
        grid=(),
    )(dn, r2, b2)


@jax.jit
def _run(X, edge_index, edge_weight, skip, H, C, p):
    src = edge_index[0].astype(jnp.int32)
    dst = edge_index[1].astype(jnp.int32)
    ew0 = edge_weight[:, 0]

    padn = lambda x: jnp.pad(x, ((0, NP - N), (0, 0)))
    xp = padn(X)
    hp = padn(H[0])
    cp = padn(C[0])
    skp = padn(skip)
    table = jnp.concatenate([xp, hp], axis=0)

    msg = _k1_msg(table, src, dst, ew0)

    wx = jnp.concatenate([p['Wx_i'], p['Wx_f'], p['Wx_c'], p['Wx_o']], axis=1)
    wh = jnp.concatenate([p['Wh_i'], p['Wh_f'], p['Wh_c'], p['Wh_o']], axis=1)
    bcat = jnp.concatenate([p['b_i'], p['b_f'], p['b_c'], p['b_o']])[None]
    wp = jnp.stack([p['wp_i'], p['wp_f'], p['wp_o']])
    lng = jnp.stack([p['ln_g_o'], p['ln_g_h'], p['ln_g_c']])
    lnb = jnp.stack([p['ln_b_o'], p['ln_b_h'], p['ln_b_c']])
    w1 = jnp.concatenate([p['Wq1'], p['Wk1'], p['Wv1'], p['Wr1']], axis=1)

    hid, cel, q_t, qe2, kv_t, r1 = _k2_call(
        xp, msg[0], hp, msg[1], cp, skp, wx, wh, bcat, wp, lng, lnb,
        w1, p['We1'])

    w2vec = jnp.pad(p['We2'][:, 0], (0, 14))
    acc, dn3, _part, e2 = _k3_tconv1(q_t, kv_t, qe2[:, 0], qe2[:, 1],
                                     src, dst, ew0, edge_weight[:, 1], w2vec)

    w2cat = jnp.concatenate([p['Wq2'], p['Wk2'], p['Wv2'], p['Wr2']], axis=1)
    qkvr2 = _k4_call(acc, dn3.reshape(3, NP).T, r1, p['We1'], p['b1'][None],
                     w2cat)

    dn = _k5_tconv2(qkvr2[:, 0], qkvr2[:, 1], qkvr2[:, 2], src, dst, e2)

    out = _k6_call(dn.reshape(64, NP), qkvr2[:, 3][None], p['b2'][None])
    return (out[0, :N, None], hid[:N][None], cel[:N][None])


def kernel(X, edge_index, edge_weight, skip, H, C, params):
    return _run(X, edge_index, edge_weight, skip, H, C, params)


# speedup vs baseline: 6.2560x; 1.1953x over previous
import functools
import numpy as np
import jax
import jax.numpy as jnp
from jax import lax
from jax.experimental import pallas as pl
from jax.experimental.pallas import tpu as pltpu
from jax.experimental.pallas import tpu_sc as plsc

N = 10000
E = 320000
D = 128
SK = 2
NP = 10240
KVW = 256
CH = 80
NR1 = 10112
HN = NP // 2
EPT1 = E // 16
EPT5 = E // 32
CH5 = 2000

_mesh = plsc.VectorSubcoreMesh(core_axis_name="c", subcore_axis_name="s")
_sc_params = pltpu.CompilerParams(needs_layout_passes=False)


def _zero_vmem(ref, rows, vregs):
    def body(r, _):
        for j in range(vregs):
            ref[r, pl.ds(j * 16, 16)] = jnp.zeros((16,), jnp.float32)
        return 0
    lax.fori_loop(0, rows, body, 0)


NR1H = 5120


@functools.partial(
    pl.kernel, mesh=_mesh, compiler_params=_sc_params,
    out_type=jax.ShapeDtypeStruct((2, NP, D), jnp.float32),
    scratch_types=[
        pltpu.VMEM((CH,), jnp.int32),
        pltpu.VMEM((CH,), jnp.int32),
        pltpu.VMEM((CH,), jnp.int32),
        pltpu.VMEM((CH,), jnp.int32),
        pltpu.VMEM((CH,), jnp.float32),
        pltpu.VMEM((CH, D), jnp.float32),
        pltpu.VMEM((CH, D), jnp.float32),
        pltpu.VMEM((64, D), jnp.float32),
        pltpu.VMEM_SHARED((NR1H, D), jnp.float32),
        pltpu.VMEM_SHARED((NR1H, D), jnp.float32),
        pltpu.SemaphoreType.DMA,
        pltpu.SemaphoreType.DMA,
    ],
)
def _k1_msg(table_hbm, src_hbm, dst_hbm, ew_hbm, msg_hbm,
            src_v, srch_v, dst_v, dloc_v, ew_v, rows_x, rows_h, buf_v,
            accx_sh, acch_sh, semi, sem):
    cid = lax.axis_index("c")
    sid = lax.axis_index("s")
    base0 = sid * EPT1
    lo1 = cid * NR1H

    _zero_vmem(buf_v, 64, D // 16)
    for zc in range(5):
        pltpu.sync_copy(buf_v, accx_sh.at[pl.ds(sid * 320 + zc * 64, 64)])
        pltpu.sync_copy(buf_v, acch_sh.at[pl.ds(sid * 320 + zc * 64, 64)])
    plsc.subcore_barrier()

    def chunk(ci, _):
        eb = base0 + ci * CH
        c1 = pltpu.async_copy(src_hbm.at[pl.ds(eb, CH)], src_v, semi)
        c2 = pltpu.async_copy(dst_hbm.at[pl.ds(eb, CH)], dst_v, semi)
        c3 = pltpu.async_copy(ew_hbm.at[pl.ds(eb, CH)], ew_v, semi)
        c1.wait()
        c2.wait()
        c3.wait()
        for g in range(CH // 16):
            srch_v[pl.ds(g * 16, 16)] = src_v[pl.ds(g * 16, 16)] + NP
            dg = dst_v[pl.ds(g * 16, 16)] - lo1
            rm = jnp.where((dg >= 0) & (dg < NR1H), 1.0, 0.0)
            dloc_v[pl.ds(g * 16, 16)] = jnp.minimum(
                jnp.maximum(dg, 0), NR1H - 1)
            ew_v[pl.ds(g * 16, 16)] = ew_v[pl.ds(g * 16, 16)] * rm
        g1 = pltpu.async_copy(table_hbm.at[src_v], rows_x, sem)
        g2 = pltpu.async_copy(table_hbm.at[srch_v], rows_h, sem)
        g1.wait()
        g2.wait()

        def group(gi, _):
            wv = ew_v[pl.ds(gi * 16, 16)]
            for e16 in range(16):
                w = wv[e16]
                r = gi * 16 + e16
                for j in range(D // 16):
                    rows_x[r, pl.ds(j * 16, 16)] = \
                        rows_x[r, pl.ds(j * 16, 16)] * w
                    rows_h[r, pl.ds(j * 16, 16)] = \
                        rows_h[r, pl.ds(j * 16, 16)] * w
            return 0
        lax.fori_loop(0, CH // 16, group, 0)
        pltpu.sync_copy(rows_x, accx_sh.at[dloc_v], add=True)
        pltpu.sync_copy(rows_h, acch_sh.at[dloc_v], add=True)
        return 0

    lax.fori_loop(0, EPT1 // CH, chunk, 0)
    plsc.subcore_barrier()
    for zc in range(5):
        rb = sid * 320 + zc * 64
        pltpu.sync_copy(accx_sh.at[pl.ds(rb, 64)], buf_v)
        pltpu.sync_copy(buf_v, msg_hbm.at[0, pl.ds(lo1 + rb, 64)])
        pltpu.sync_copy(acch_sh.at[pl.ds(rb, 64)], buf_v)
        pltpu.sync_copy(buf_v, msg_hbm.at[1, pl.ds(lo1 + rb, 64)])


@functools.partial(
    pl.kernel, mesh=_mesh, compiler_params=_sc_params,
    out_type=[jax.ShapeDtypeStruct((NP, D), jnp.float32),
              jax.ShapeDtypeStruct((3 * NP,), jnp.float32),
              jax.ShapeDtypeStruct((96 * HN,), jnp.float32),
              jax.ShapeDtypeStruct((E,), jnp.float32)],
    scratch_types=[
        pltpu.VMEM((CH,), jnp.int32),
        pltpu.VMEM((CH,), jnp.int32),
        pltpu.VMEM((CH,), jnp.int32),
        pltpu.VMEM((CH,), jnp.float32),
        pltpu.VMEM((CH,), jnp.float32),
        pltpu.VMEM((CH,), jnp.float32),
        pltpu.VMEM((CH, D), jnp.float32),
        pltpu.VMEM((CH, KVW), jnp.float32),
        pltpu.VMEM((CH, D), jnp.float32),
        pltpu.VMEM((64, D), jnp.float32),
        pltpu.VMEM((16,), jnp.float32),
        pltpu.VMEM((HN,), jnp.float32),
        pltpu.VMEM((HN,), jnp.float32),
        pltpu.VMEM((HN,), jnp.float32),
        pltpu.VMEM((HN,), jnp.float32),
        pltpu.VMEM((HN,), jnp.float32),
        pltpu.VMEM((320,), jnp.float32),
        pltpu.VMEM((320,), jnp.float32),
        pltpu.VMEM_SHARED((HN, D), jnp.float32),
        pltpu.SemaphoreType.DMA,
    ],
)
def _k3_tconv1(q_hbm, kv_hbm, qe0_hbm, qe1_hbm, src_hbm, dst_hbm,
               ew0_hbm, ew1_hbm, w2_hbm,
               acc_hbm, dn3_hbm, part_hbm, e2_hbm,
               src_v, dst_v, dloc_v, ew0_v, ew1_v, e2_v, qrows, kvrows, srows,
               buf_v, w2_v, qe0_v, qe1_v, den_t, dw0_t, dw1_t, tmp_v, red_v,
               acc_sh, sem):
    cid = lax.axis_index("c")
    sid = lax.axis_index("s")
    lo = cid * HN
    _zero_vmem(buf_v, 64, D // 16)
    for zc in range(5):
        pltpu.sync_copy(buf_v, acc_sh.at[pl.ds(sid * 320 + zc * 64, 64)])
    pltpu.sync_copy(w2_hbm, w2_v)
    pltpu.sync_copy(qe0_hbm.at[pl.ds(lo, HN)], qe0_v)
    pltpu.sync_copy(qe1_hbm.at[pl.ds(lo, HN)], qe1_v)

    def zs(r, _):
        den_t[pl.ds(r * 16, 16)] = jnp.zeros((16,), jnp.float32)
        dw0_t[pl.ds(r * 16, 16)] = jnp.zeros((16,), jnp.float32)
        dw1_t[pl.ds(r * 16, 16)] = jnp.zeros((16,), jnp.float32)
        return 0
    lax.fori_loop(0, HN // 16, zs, 0)
    plsc.subcore_barrier()

    wid = sid * 2 + cid
    base0 = sid * EPT1
    w2g = w2_v[...]
    w20 = w2g[0]
    w21 = w2g[1]
    lanes = lax.iota(jnp.int32, 16)
    z16 = jnp.zeros((16,), jnp.int32)
    perms = [lanes ^ (1 << k) for k in range(4)]

    dnums = lax.GatherDimensionNumbers(
        offset_dims=(), collapsed_slice_dims=(0,), start_index_map=(0,))

    def _lanesum(d):
        for p_ in perms:
            d = d + lax.gather(
                d, p_[:, None], dnums, slice_sizes=(1,),
                mode=lax.GatherScatterMode.PROMISE_IN_BOUNDS,
                unique_indices=True, indices_are_sorted=False)
        return d

    def chunk(ci, _):
        eb = base0 + ci * CH
        pltpu.sync_copy(src_hbm.at[pl.ds(eb, CH)], src_v)
        pltpu.sync_copy(dst_hbm.at[pl.ds(eb, CH)], dst_v)
        pltpu.sync_copy(ew0_hbm.at[pl.ds(eb, CH)], ew0_v)
        pltpu.sync_copy(ew1_hbm.at[pl.ds(eb, CH)], ew1_v)
        cp1 = pltpu.async_copy(q_hbm.at[dst_v], qrows, sem)
        cp2 = pltpu.async_copy(kv_hbm.at[src_v], kvrows, sem)
        cp1.wait()
        cp2.wait()

        def group(gi, _):
            ewv0 = ew0_v[pl.ds(gi * 16, 16)]
            ewv1 = ew1_v[pl.ds(gi * 16, 16)]
            e2_v[pl.ds(gi * 16, 16)] = ewv0 * w20 + ewv1 * w21
            di16 = dst_v[pl.ds(gi * 16, 16)]
            dil = di16 - lo
            rm = jnp.where((dil >= 0) & (dil < HN), 1.0, 0.0).astype(jnp.float32)
            dil = jnp.minimum(jnp.maximum(dil, 0), HN - 1)
            dloc_v[pl.ds(gi * 16, 16)] = dil
            qe0g = plsc.load_gather(qe0_v, [dil])
            qe1g = plsc.load_gather(qe1_v, [dil])
            aext = qe0g * ewv0 + qe1g * ewv1
            evec = jnp.zeros((16,), jnp.float32)
            for e16 in range(16):
                r = gi * 16 + e16
                d = qrows[r, pl.ds(0, 16)] * kvrows[r, pl.ds(0, 16)]
                for j in range(1, D // 16):
                    d = d + qrows[r, pl.ds(j * 16, 16)] * kvrows[r, pl.ds(j * 16, 16)]
                sv = _lanesum(d)
                exm = jnp.exp(sv + aext[e16]) * rm[e16]
                for j in range(D // 16):
                    srows[r, pl.ds(j * 16, 16)] = kvrows[r, pl.ds(D + j * 16, 16)] * exm
                evec = jnp.where(lanes == e16, exm, evec)
            plsc.addupdate_scatter(den_t, [dil], evec)
            plsc.addupdate_scatter(dw0_t, [dil], evec * ewv0)
            plsc.addupdate_scatter(dw1_t, [dil], evec * ewv1)
            return 0
        lax.fori_loop(0, CH // 16, group, 0)

        @pl.when(cid == 0)
        def _():
            pltpu.sync_copy(e2_v, e2_hbm.at[pl.ds(eb, CH)])
        pltpu.sync_copy(srows, acc_sh.at[dloc_v], add=True)
        return 0

    lax.fori_loop(0, EPT1 // CH, chunk, 0)

    pltpu.sync_copy(den_t, part_hbm.at[pl.ds((wid * 3) * HN, HN)])
    pltpu.sync_copy(dw0_t, part_hbm.at[pl.ds((wid * 3 + 1) * HN, HN)])
    pltpu.sync_copy(dw1_t, part_hbm.at[pl.ds((wid * 3 + 2) * HN, HN)])
    plsc.subcore_barrier()
    for zc in range(5):
        rb = sid * 320 + zc * 64
        pltpu.sync_copy(acc_sh.at[pl.ds(rb, 64)], buf_v)
        pltpu.sync_copy(buf_v, acc_hbm.at[pl.ds(lo + rb, 64)])
    rb = sid * 320
    for half in range(3):
        def rz(g, _):
            red_v[pl.ds(g * 16, 16)] = jnp.zeros((16,), jnp.float32)
            return 0
        lax.fori_loop(0, 320 // 16, rz, 0)

        def racc(t, _):
            pltpu.sync_copy(
                part_hbm.at[pl.ds(((t * 2 + cid) * 3 + half) * HN + rb, 320)],
                tmp_v)

            def radd(g, _):
                red_v[pl.ds(g * 16, 16)] = (red_v[pl.ds(g * 16, 16)]
                                            + tmp_v[pl.ds(g * 16, 16)])
                return 0
            lax.fori_loop(0, 320 // 16, radd, 0)
            return 0
        lax.fori_loop(0, 16, racc, 0)
        pltpu.sync_copy(red_v,
                        dn3_hbm.at[pl.ds(half * NP + lo + rb, 320)])


@functools.partial(
    pl.kernel, mesh=_mesh, compiler_params=_sc_params,
    out_type=jax.ShapeDtypeStruct((64 * NP,), jnp.float32),
    scratch_types=[
        pltpu.VMEM((NP,), jnp.float32),
        pltpu.VMEM((NP,), jnp.float32),
        pltpu.VMEM((NP,), jnp.float32),
        pltpu.VMEM((NP,), jnp.float32),
        pltpu.VMEM((NP,), jnp.float32),
        pltpu.VMEM((CH5,), jnp.int32),
        pltpu.VMEM((CH5,), jnp.int32),
        pltpu.VMEM((CH5,), jnp.float32),
        pltpu.SemaphoreType.DMA,
    ],
)
def _k5_tconv2(q2_hbm, k2_hbm, v2_hbm, src_hbm, dst_hbm, e2_hbm, dn_hbm,
               q2_v, k2_v, v2_v, den_t, num_t, src_v, dst_v, e2_v, sem):
    cid = lax.axis_index("c")
    sid = lax.axis_index("s")
    pltpu.sync_copy(q2_hbm, q2_v)
    pltpu.sync_copy(k2_hbm, k2_v)
    pltpu.sync_copy(v2_hbm, v2_v)

    def zbody(r, _):
        den_t[pl.ds(r * 16, 16)] = jnp.zeros((16,), jnp.float32)
        num_t[pl.ds(r * 16, 16)] = jnp.zeros((16,), jnp.float32)
        return 0
    lax.fori_loop(0, NP // 16, zbody, 0)

    wid = sid * 2 + cid
    base0 = wid * EPT5
    z16 = jnp.zeros((16,), jnp.int32)

    def chunk(ci, _):
        eb = base0 + ci * CH5
        pltpu.sync_copy(src_hbm.at[pl.ds(eb, CH5)], src_v)
        pltpu.sync_copy(dst_hbm.at[pl.ds(eb, CH5)], dst_v)
        pltpu.sync_copy(e2_hbm.at[pl.ds(eb, CH5)], e2_v)

        def group(g, _):
            si = src_v[pl.ds(g * 16, 16)]
            di = dst_v[pl.ds(g * 16, 16)]
            e2g = e2_v[pl.ds(g * 16, 16)]
            qv = plsc.load_gather(q2_v, [di])
            kv = plsc.load_gather(k2_v, [si])
            vv = plsc.load_gather(v2_v, [si])
            ex = jnp.exp(qv * (kv + e2g))
            plsc.addupdate_scatter(den_t, [di], ex)
            plsc.addupdate_scatter(num_t, [di], ex * (vv + e2g))
            return 0
        lax.fori_loop(0, CH5 // 16, group, 0)
        return 0
    lax.fori_loop(0, EPT5 // CH5, chunk, 0)

    pltpu.sync_copy(den_t, dn_hbm.at[pl.ds((wid * 2) * NP, NP)])
    pltpu.sync_copy(num_t, dn_hbm.at[pl.ds((wid * 2 + 1) * NP, NP)])


def _ln_rows(x, g, b):
    m = jnp.mean(x, axis=-1, keepdims=True)
    v = jnp.mean((x - m) * (x - m), axis=-1, keepdims=True)
    return (x - m) / jnp.sqrt(v + 1e-5) * g + b


def _k2_body(x_ref, mx_ref, h_ref, mh_ref, c_ref, sk_ref,
             wx_ref, wh_ref, bcat_ref, wp_ref, lng_ref, lnb_ref,
             w1_ref, we1_ref,
             hid_ref, cel_ref, q_ref, qe2_ref, kv_ref, r_ref):
    a = x_ref[:] + mx_ref[:]
    bh = h_ref[:] + mh_ref[:]
    c = c_ref[:]
    z = (jnp.dot(a, wx_ref[:], preferred_element_type=jnp.float32)
         + jnp.dot(bh, wh_ref[:], preferred_element_type=jnp.float32)
         + bcat_ref[:])
    zi = z[:, 0:D]
    zf = z[:, D:2 * D]
    zg = z[:, 2 * D:3 * D]
    zo = z[:, 3 * D:4 * D]
    wpi = wp_ref[0:1, :]
    wpf = wp_ref[1:2, :]
    wpo = wp_ref[2:3, :]
    ig = jax.nn.sigmoid(zi + wpi * c)
    fg = jax.nn.sigmoid(zf + wpf * c)
    gg = jnp.tanh(zg)
    cn = fg * c + ig * gg
    og = jax.nn.sigmoid(zo + wpo * cn)
    hn = og * jnp.tanh(cn)
    hid_ref[:] = _ln_rows(hn, lng_ref[1:2, :], lnb_ref[1:2, :])
    cel_ref[:] = _ln_rows(cn, lng_ref[2:3, :], lnb_ref[2:3, :])
    o0 = jax.nn.relu(_ln_rows(hn, lng_ref[0:1, :], lnb_ref[0:1, :]))
    x130 = jnp.concatenate([o0, sk_ref[:]], axis=1)
    qkvr = jnp.dot(x130, w1_ref[:], preferred_element_type=jnp.float32)
    inv = np.float32(1.0 / np.sqrt(float(D)))
    q = qkvr[:, 0:D] * inv
    r_ref[:] = qkvr[:, 3 * D:4 * D]
    q_ref[:] = q
    qe2_ref[:] = jnp.dot(q, we1_ref[:].T, preferred_element_type=jnp.float32)
    kv_ref[:] = qkvr[:, D:3 * D]


def _k2_call(xp, mx, hp, mh, cp, skp, wx, wh, bcat, wp, lng, lnb, w1, we1):
    BLK = 512
    grid = (NP // BLK,)
    row = lambda i: (i, 0)
    full = lambda i: (0, 0)
    return pl.pallas_call(
        _k2_body,
        grid=grid,
        in_specs=[
            pl.BlockSpec((BLK, D), row), pl.BlockSpec((BLK, D), row),
            pl.BlockSpec((BLK, D), row), pl.BlockSpec((BLK, D), row),
            pl.BlockSpec((BLK, D), row), pl.BlockSpec((BLK, SK), row),
            pl.BlockSpec((D, 4 * D), full), pl.BlockSpec((D, 4 * D), full),
            pl.BlockSpec((1, 4 * D), full), pl.BlockSpec((3, D), full),
            pl.BlockSpec((3, D), full), pl.BlockSpec((3, D), full),
            pl.BlockSpec((D + SK, 4 * D), full), pl.BlockSpec((2, D), full),
        ],
        out_specs=[
            pl.BlockSpec((BLK, D), row), pl.BlockSpec((BLK, D), row),
            pl.BlockSpec((BLK, D), row), pl.BlockSpec((BLK, 2), row),
            pl.BlockSpec((BLK, KVW), row), pl.BlockSpec((BLK, D), row),
        ],
        out_shape=[
            jax.ShapeDtypeStruct((NP, D), jnp.float32),
            jax.ShapeDtypeStruct((NP, D), jnp.float32),
            jax.ShapeDtypeStruct((NP, D), jnp.float32),
            jax.ShapeDtypeStruct((NP, 2), jnp.float32),
            jax.ShapeDtypeStruct((NP, KVW), jnp.float32),
            jax.ShapeDtypeStruct((NP, D), jnp.float32),
        ],
    )(xp, mx, hp, mh, cp, skp, wx, wh, bcat, wp, lng, lnb, w1, we1)


def _k4_body(acc_ref, dst_ref, r_ref, we1_ref, b1_ref, w2_ref, qkvr2_ref):
    numv = acc_ref[:]
    st = dst_ref[:]
    den = st[:, 0:1]
    dew = st[:, 1:3]
    agg = (numv + jnp.dot(dew, we1_ref[:], preferred_element_type=jnp.float32)) \
        / (den + 1e-16)
    out1 = jax.nn.relu(agg + r_ref[:] + b1_ref[:])
    qkvr2_ref[:] = jnp.dot(out1, w2_ref[:], preferred_element_type=jnp.float32)


def _k4_call(acc, dstats_t, r1, we1, b1, w2):
    BLK = 512
    grid = (NP // BLK,)
    return pl.pallas_call(
        _k4_body,
        grid=grid,
        in_specs=[
            pl.BlockSpec((BLK, D), lambda i: (i, 0)),
            pl.BlockSpec((BLK, 3), lambda i: (i, 0)),
            pl.BlockSpec((BLK, D), lambda i: (i, 0)),
            pl.BlockSpec((2, D), lambda i: (0, 0)),
            pl.BlockSpec((1, D), lambda i: (0, 0)),
            pl.BlockSpec((D, 4), lambda i: (0, 0)),
        ],
        out_specs=pl.BlockSpec((BLK, 4), lambda i: (i, 0)),
        out_shape=jax.ShapeDtypeStruct((NP, 4), jnp.float32),
    )(acc, dstats_t, r1, we1, b1, w2)


def _k6_body(dn_ref, r2_ref, b2_ref, out_ref):
    den = dn_ref[0:1, :]
    num = dn_ref[1:2, :]
    for t in range(1, 32):
        den = den + dn_ref[2 * t:2 * t + 1, :]
        num = num + dn_ref[2 * t + 1:2 * t + 2, :]
    out_ref[:] = jax.nn.sigmoid(num / (den + 1e-16) + r2_ref[:] + b2_ref[0, 0])


def _k6_call(dn, r2, b2):
    return pl.pallas_call(
        _k6_body,
        in_specs=[
            pl.BlockSpec((64, NP), lambda: (0, 0)),
            pl.BlockSpec((1, NP), lambda: (0, 0)),
            pl.BlockSpec((1, 1), lambda: (0, 0)),
        ],
        out_specs=pl.BlockSpec((1, NP), lambda: (0, 0)),
        out_shape=jax.ShapeDtypeStruct((1, NP), jnp.float32),
        grid=(),
    )(dn, r2, b2)


@jax.jit
def _run(X, edge_index, edge_weight, skip, H, C, p):
    src = edge_index[0].astype(jnp.int32)
    dst = edge_index[1].astype(jnp.int32)
    ew0 = edge_weight[:, 0]

    padn = lambda x: jnp.pad(x, ((0, NP - N), (0, 0)))
    xp = padn(X)
    hp = padn(H[0])
    cp = padn(C[0])
    skp = padn(skip)
    table = jnp.concatenate([xp, hp], axis=0)

    msg = _k1_msg(table, src, dst, ew0)

    wx = jnp.concatenate([p['Wx_i'], p['Wx_f'], p['Wx_c'], p['Wx_o']], axis=1)
    wh = jnp.concatenate([p['Wh_i'], p['Wh_f'], p['Wh_c'], p['Wh_o']], axis=1)
    bcat = jnp.concatenate([p['b_i'], p['b_f'], p['b_c'], p['b_o']])[None]
    wp = jnp.stack([p['wp_i'], p['wp_f'], p['wp_o']])
    lng = jnp.stack([p['ln_g_o'], p['ln_g_h'], p['ln_g_c']])
    lnb = jnp.stack([p['ln_b_o'], p['ln_b_h'], p['ln_b_c']])
    w1 = jnp.concatenate([p['Wq1'], p['Wk1'], p['Wv1'], p['Wr1']], axis=1)

    hid, cel, q_t, qe2, kv_t, r1 = _k2_call(
        xp, msg[0], hp, msg[1], cp, skp, wx, wh, bcat, wp, lng, lnb,
        w1, p['We1'])

    w2vec = jnp.pad(p['We2'][:, 0], (0, 14))
    acc, dn3, _part, e2 = _k3_tconv1(q_t, kv_t, qe2[:, 0], qe2[:, 1],
                                     src, dst, ew0, edge_weight[:, 1], w2vec)

    w2cat = jnp.concatenate([p['Wq2'], p['Wk2'], p['Wv2'], p['Wr2']], axis=1)
    qkvr2 = _k4_call(acc, dn3.reshape(3, NP).T, r1, p['We1'], p['b1'][None],
                     w2cat)

    dn = _k5_tconv2(qkvr2[:, 0], qkvr2[:, 1], qkvr2[:, 2], src, dst, e2)

    out = _k6_call(dn.reshape(64, NP), qkvr2[:, 3][None], p['b2'][None])
    return (out[0, :N, None], hid[:N][None], cel[:N][None])


def kernel(X, edge_index, edge_weight, skip, H, C, params):
    return _run(X, edge_index, edge_weight, skip, H, C, params)


# speedup vs baseline: 7.2913x; 1.1655x over previous
import functools
import numpy as np
import jax
import jax.numpy as jnp
from jax import lax
from jax.experimental import pallas as pl
from jax.experimental.pallas import tpu as pltpu
from jax.experimental.pallas import tpu_sc as plsc

N = 10000
E = 320000
D = 128
SK = 2
NP = 10240
KVW = 256
CH = 80
NR1 = 10112
HN = NP // 2
EPT1 = E // 16
EPT5 = E // 32
CH5 = 2000

_mesh = plsc.VectorSubcoreMesh(core_axis_name="c", subcore_axis_name="s")
_sc_params = pltpu.CompilerParams(needs_layout_passes=False)


def _zero_vmem(ref, rows, vregs):
    def body(r, _):
        for j in range(vregs):
            ref[r, pl.ds(j * 16, 16)] = jnp.zeros((16,), jnp.float32)
        return 0
    lax.fori_loop(0, rows, body, 0)


NR1H = 5120
CHK = 128
K1TAIL = EPT1 - (EPT1 // CHK) * CHK


@functools.partial(
    pl.kernel, mesh=_mesh, compiler_params=_sc_params,
    out_type=jax.ShapeDtypeStruct((2, NP, D), jnp.float32),
    scratch_types=[
        pltpu.VMEM((CHK,), jnp.int32),
        pltpu.VMEM((CHK,), jnp.int32),
        pltpu.VMEM((CHK,), jnp.int32),
        pltpu.VMEM((CHK,), jnp.int32),
        pltpu.VMEM((K1TAIL,), jnp.int32),
        pltpu.VMEM((CHK,), jnp.float32),
        pltpu.VMEM((CHK, D), jnp.float32),
        pltpu.VMEM((CHK, D), jnp.float32),
        pltpu.VMEM((64, D), jnp.float32),
        pltpu.VMEM_SHARED((NR1H, D), jnp.float32),
        pltpu.VMEM_SHARED((NR1H, D), jnp.float32),
        pltpu.SemaphoreType.DMA,
        pltpu.SemaphoreType.DMA,
    ],
)
def _k1_msg(table_hbm, src_hbm, dst_hbm, ew_hbm, msg_hbm,
            src_v, srch_v, dst_v, dloc_v, dloct_v, ew_v, rows_x, rows_h,
            buf_v, accx_sh, acch_sh, semi, sem):
    cid = lax.axis_index("c")
    sid = lax.axis_index("s")
    base0 = sid * EPT1
    lo1 = cid * NR1H

    _zero_vmem(buf_v, 64, D // 16)
    for zc in range(5):
        pltpu.sync_copy(buf_v, accx_sh.at[pl.ds(sid * 320 + zc * 64, 64)])
        pltpu.sync_copy(buf_v, acch_sh.at[pl.ds(sid * 320 + zc * 64, 64)])
    plsc.subcore_barrier()

    def do_chunk(eb, n, dloc_ref):
        c1 = pltpu.async_copy(src_hbm.at[pl.ds(eb, n)],
                              src_v.at[pl.ds(0, n)], semi)
        c2 = pltpu.async_copy(dst_hbm.at[pl.ds(eb, n)],
                              dst_v.at[pl.ds(0, n)], semi)
        c3 = pltpu.async_copy(ew_hbm.at[pl.ds(eb, n)],
                              ew_v.at[pl.ds(0, n)], semi)
        c1.wait()
        c2.wait()
        c3.wait()
        for g in range(n // 16):
            srch_v[pl.ds(g * 16, 16)] = src_v[pl.ds(g * 16, 16)] + NP
            dg = dst_v[pl.ds(g * 16, 16)] - lo1
            rm = jnp.where((dg >= 0) & (dg < NR1H), 1.0, 0.0)
            dloc_ref[pl.ds(g * 16, 16)] = jnp.minimum(
                jnp.maximum(dg, 0), NR1H - 1)
            ew_v[pl.ds(g * 16, 16)] = ew_v[pl.ds(g * 16, 16)] * rm
        g1 = pltpu.async_copy(table_hbm.at[src_v.at[pl.ds(0, n)]],
                              rows_x.at[pl.ds(0, n)], sem)
        g2 = pltpu.async_copy(table_hbm.at[srch_v.at[pl.ds(0, n)]],
                              rows_h.at[pl.ds(0, n)], sem)
        g1.wait()
        g2.wait()

        def group(gi, _):
            wv = ew_v[pl.ds(gi * 16, 16)]
            for e16 in range(16):
                w = wv[e16]
                r = gi * 16 + e16
                for j in range(D // 16):
                    rows_x[r, pl.ds(j * 16, 16)] = \
                        rows_x[r, pl.ds(j * 16, 16)] * w
                    rows_h[r, pl.ds(j * 16, 16)] = \
                        rows_h[r, pl.ds(j * 16, 16)] * w
            return 0
        lax.fori_loop(0, n // 16, group, 0)
        s1 = pltpu.async_copy(rows_x.at[pl.ds(0, n)],
                              accx_sh.at[dloc_ref], sem, add=True)
        s2 = pltpu.async_copy(rows_h.at[pl.ds(0, n)],
                              acch_sh.at[dloc_ref], sem, add=True)
        s1.wait()
        s2.wait()

    def chunk(ci, _):
        do_chunk(base0 + ci * CHK, CHK, dloc_v)
        return 0
    lax.fori_loop(0, EPT1 // CHK, chunk, 0)
    do_chunk(base0 + (EPT1 // CHK) * CHK, K1TAIL, dloct_v)

    plsc.subcore_barrier()
    for zc in range(5):
        rb = sid * 320 + zc * 64
        pltpu.sync_copy(accx_sh.at[pl.ds(rb, 64)], buf_v)
        pltpu.sync_copy(buf_v, msg_hbm.at[0, pl.ds(lo1 + rb, 64)])
        pltpu.sync_copy(acch_sh.at[pl.ds(rb, 64)], buf_v)
        pltpu.sync_copy(buf_v, msg_hbm.at[1, pl.ds(lo1 + rb, 64)])


CH3 = 112
K3TAIL = EPT1 - (EPT1 // CH3) * CH3


@functools.partial(
    pl.kernel, mesh=_mesh, compiler_params=_sc_params,
    out_type=[jax.ShapeDtypeStruct((NP, D), jnp.float32),
              jax.ShapeDtypeStruct((3 * NP,), jnp.float32),
              jax.ShapeDtypeStruct((96 * HN,), jnp.float32),
              jax.ShapeDtypeStruct((E,), jnp.float32)],
    scratch_types=[
        pltpu.VMEM((CH3,), jnp.int32),
        pltpu.VMEM((CH3,), jnp.int32),
        pltpu.VMEM((CH3,), jnp.int32),
        pltpu.VMEM((K3TAIL,), jnp.int32),
        pltpu.VMEM((CH3,), jnp.float32),
        pltpu.VMEM((CH3,), jnp.float32),
        pltpu.VMEM((CH3,), jnp.float32),
        pltpu.VMEM((CH3, D), jnp.float32),
        pltpu.VMEM((CH3, KVW), jnp.float32),
        pltpu.VMEM((CH3, D), jnp.float32),
        pltpu.VMEM((16, D), jnp.float32),
        pltpu.VMEM((16,), jnp.float32),
        pltpu.VMEM((HN,), jnp.float32),
        pltpu.VMEM((HN,), jnp.float32),
        pltpu.VMEM((HN,), jnp.float32),
        pltpu.VMEM((HN,), jnp.float32),
        pltpu.VMEM((HN,), jnp.float32),
        pltpu.VMEM((320,), jnp.float32),
        pltpu.VMEM((320,), jnp.float32),
        pltpu.VMEM_SHARED((HN, D), jnp.float32),
        pltpu.SemaphoreType.DMA,
        pltpu.SemaphoreType.DMA,
    ],
)
def _k3_tconv1(q_hbm, kv_hbm, qe0_hbm, qe1_hbm, src_hbm, dst_hbm,
               ew0_hbm, ew1_hbm, w2_hbm,
               acc_hbm, dn3_hbm, part_hbm, e2_hbm,
               src_v, dst_v, dloc_v, dloct_v, ew0_v, ew1_v, e2_v,
               qrows, kvrows, srows, buf_v, w2_v, qe0_v, qe1_v,
               den_t, dw0_t, dw1_t, tmp_v, red_v, acc_sh, semi, sem):
    cid = lax.axis_index("c")
    sid = lax.axis_index("s")
    lo = cid * HN
    _zero_vmem(buf_v, 16, D // 16)
    for zc in range(20):
        pltpu.sync_copy(buf_v, acc_sh.at[pl.ds(sid * 320 + zc * 16, 16)])
    pltpu.sync_copy(w2_hbm, w2_v)
    pltpu.sync_copy(qe0_hbm.at[pl.ds(lo, HN)], qe0_v)
    pltpu.sync_copy(qe1_hbm.at[pl.ds(lo, HN)], qe1_v)

    def zs(r, _):
        den_t[pl.ds(r * 16, 16)] = jnp.zeros((16,), jnp.float32)
        dw0_t[pl.ds(r * 16, 16)] = jnp.zeros((16,), jnp.float32)
        dw1_t[pl.ds(r * 16, 16)] = jnp.zeros((16,), jnp.float32)
        return 0
    lax.fori_loop(0, HN // 16, zs, 0)
    plsc.subcore_barrier()

    wid = sid * 2 + cid
    base0 = sid * EPT1
    w2g = w2_v[...]
    w20 = w2g[0]
    w21 = w2g[1]
    lanes = lax.iota(jnp.int32, 16)
    perms = [lanes ^ (1 << k) for k in range(4)]

    dnums = lax.GatherDimensionNumbers(
        offset_dims=(), collapsed_slice_dims=(0,), start_index_map=(0,))

    def _lanesum(d):
        for p_ in perms:
            d = d + lax.gather(
                d, p_[:, None], dnums, slice_sizes=(1,),
                mode=lax.GatherScatterMode.PROMISE_IN_BOUNDS,
                unique_indices=True, indices_are_sorted=False)
        return d

    def do_chunk(eb, n, dloc_ref):
        c1 = pltpu.async_copy(src_hbm.at[pl.ds(eb, n)],
                              src_v.at[pl.ds(0, n)], semi)
        c2 = pltpu.async_copy(dst_hbm.at[pl.ds(eb, n)],
                              dst_v.at[pl.ds(0, n)], semi)
        c3 = pltpu.async_copy(ew0_hbm.at[pl.ds(eb, n)],
                              ew0_v.at[pl.ds(0, n)], semi)
        c4 = pltpu.async_copy(ew1_hbm.at[pl.ds(eb, n)],
                              ew1_v.at[pl.ds(0, n)], semi)
        c1.wait()
        c2.wait()
        c3.wait()
        c4.wait()
        cp1 = pltpu.async_copy(q_hbm.at[dst_v.at[pl.ds(0, n)]],
                               qrows.at[pl.ds(0, n)], sem)
        cp2 = pltpu.async_copy(kv_hbm.at[src_v.at[pl.ds(0, n)]],
                               kvrows.at[pl.ds(0, n)], sem)
        cp1.wait()
        cp2.wait()

        def group(gi, _):
            ewv0 = ew0_v[pl.ds(gi * 16, 16)]
            ewv1 = ew1_v[pl.ds(gi * 16, 16)]
            e2_v[pl.ds(gi * 16, 16)] = ewv0 * w20 + ewv1 * w21
            di16 = dst_v[pl.ds(gi * 16, 16)]
            dil = di16 - lo
            rm = jnp.where((dil >= 0) & (dil < HN), 1.0, 0.0).astype(jnp.float32)
            dil = jnp.minimum(jnp.maximum(dil, 0), HN - 1)
            dloc_ref[pl.ds(gi * 16, 16)] = dil
            qe0g = plsc.load_gather(qe0_v, [dil])
            qe1g = plsc.load_gather(qe1_v, [dil])
            aext = qe0g * ewv0 + qe1g * ewv1
            evec = jnp.zeros((16,), jnp.float32)
            for e16 in range(16):
                r = gi * 16 + e16
                d = qrows[r, pl.ds(0, 16)] * kvrows[r, pl.ds(0, 16)]
                for j in range(1, D // 16):
                    d = d + qrows[r, pl.ds(j * 16, 16)] * kvrows[r, pl.ds(j * 16, 16)]
                sv = _lanesum(d)
                exm = jnp.exp(sv + aext[e16]) * rm[e16]
                for j in range(D // 16):
                    srows[r, pl.ds(j * 16, 16)] = kvrows[r, pl.ds(D + j * 16, 16)] * exm
                evec = jnp.where(lanes == e16, exm, evec)
            plsc.addupdate_scatter(den_t, [dil], evec)
            plsc.addupdate_scatter(dw0_t, [dil], evec * ewv0)
            plsc.addupdate_scatter(dw1_t, [dil], evec * ewv1)
            return 0
        lax.fori_loop(0, n // 16, group, 0)

        s1 = pltpu.async_copy(srows.at[pl.ds(0, n)],
                              acc_sh.at[dloc_ref], sem, add=True)

        @pl.when(cid == 0)
        def _():
            pltpu.sync_copy(e2_v.at[pl.ds(0, n)], e2_hbm.at[pl.ds(eb, n)])
        s1.wait()

    def chunk(ci, _):
        do_chunk(base0 + ci * CH3, CH3, dloc_v)
        return 0
    lax.fori_loop(0, EPT1 // CH3, chunk, 0)
    do_chunk(base0 + (EPT1 // CH3) * CH3, K3TAIL, dloct_v)

    pltpu.sync_copy(den_t, part_hbm.at[pl.ds((wid * 3) * HN, HN)])
    pltpu.sync_copy(dw0_t, part_hbm.at[pl.ds((wid * 3 + 1) * HN, HN)])
    pltpu.sync_copy(dw1_t, part_hbm.at[pl.ds((wid * 3 + 2) * HN, HN)])
    plsc.subcore_barrier()
    for zc in range(20):
        rb = sid * 320 + zc * 16
        pltpu.sync_copy(acc_sh.at[pl.ds(rb, 16)], buf_v)
        pltpu.sync_copy(buf_v, acc_hbm.at[pl.ds(lo + rb, 16)])
    rb = sid * 320
    for half in range(3):
        def rz(g, _):
            red_v[pl.ds(g * 16, 16)] = jnp.zeros((16,), jnp.float32)
            return 0
        lax.fori_loop(0, 320 // 16, rz, 0)

        def racc(t, _):
            pltpu.sync_copy(
                part_hbm.at[pl.ds(((t * 2 + cid) * 3 + half) * HN + rb, 320)],
                tmp_v)

            def radd(g, _):
                red_v[pl.ds(g * 16, 16)] = (red_v[pl.ds(g * 16, 16)]
                                            + tmp_v[pl.ds(g * 16, 16)])
                return 0
            lax.fori_loop(0, 320 // 16, radd, 0)
            return 0
        lax.fori_loop(0, 16, racc, 0)
        pltpu.sync_copy(red_v,
                        dn3_hbm.at[pl.ds(half * NP + lo + rb, 320)])


@functools.partial(
    pl.kernel, mesh=_mesh, compiler_params=_sc_params,
    out_type=jax.ShapeDtypeStruct((64 * NP,), jnp.float32),
    scratch_types=[
        pltpu.VMEM((NP,), jnp.float32),
        pltpu.VMEM((NP,), jnp.float32),
        pltpu.VMEM((NP,), jnp.float32),
        pltpu.VMEM((NP,), jnp.float32),
        pltpu.VMEM((NP,), jnp.float32),
        pltpu.VMEM((CH5,), jnp.int32),
        pltpu.VMEM((CH5,), jnp.int32),
        pltpu.VMEM((CH5,), jnp.float32),
        pltpu.SemaphoreType.DMA,
    ],
)
def _k5_tconv2(q2_hbm, k2_hbm, v2_hbm, src_hbm, dst_hbm, e2_hbm, dn_hbm,
               q2_v, k2_v, v2_v, den_t, num_t, src_v, dst_v, e2_v, sem):
    cid = lax.axis_index("c")
    sid = lax.axis_index("s")
    pltpu.sync_copy(q2_hbm, q2_v)
    pltpu.sync_copy(k2_hbm, k2_v)
    pltpu.sync_copy(v2_hbm, v2_v)

    def zbody(r, _):
        den_t[pl.ds(r * 16, 16)] = jnp.zeros((16,), jnp.float32)
        num_t[pl.ds(r * 16, 16)] = jnp.zeros((16,), jnp.float32)
        return 0
    lax.fori_loop(0, NP // 16, zbody, 0)

    wid = sid * 2 + cid
    base0 = wid * EPT5
    z16 = jnp.zeros((16,), jnp.int32)

    def chunk(ci, _):
        eb = base0 + ci * CH5
        pltpu.sync_copy(src_hbm.at[pl.ds(eb, CH5)], src_v)
        pltpu.sync_copy(dst_hbm.at[pl.ds(eb, CH5)], dst_v)
        pltpu.sync_copy(e2_hbm.at[pl.ds(eb, CH5)], e2_v)

        def group(g, _):
            si = src_v[pl.ds(g * 16, 16)]
            di = dst_v[pl.ds(g * 16, 16)]
            e2g = e2_v[pl.ds(g * 16, 16)]
            qv = plsc.load_gather(q2_v, [di])
            kv = plsc.load_gather(k2_v, [si])
            vv = plsc.load_gather(v2_v, [si])
            ex = jnp.exp(qv * (kv + e2g))
            plsc.addupdate_scatter(den_t, [di], ex)
            plsc.addupdate_scatter(num_t, [di], ex * (vv + e2g))
            return 0
        lax.fori_loop(0, CH5 // 16, group, 0)
        return 0
    lax.fori_loop(0, EPT5 // CH5, chunk, 0)

    pltpu.sync_copy(den_t, dn_hbm.at[pl.ds((wid * 2) * NP, NP)])
    pltpu.sync_copy(num_t, dn_hbm.at[pl.ds((wid * 2 + 1) * NP, NP)])


def _ln_rows(x, g, b):
    m = jnp.mean(x, axis=-1, keepdims=True)
    v = jnp.mean((x - m) * (x - m), axis=-1, keepdims=True)
    return (x - m) / jnp.sqrt(v + 1e-5) * g + b


def _k2_body(x_ref, mx_ref, h_ref, mh_ref, c_ref, sk_ref,
             wx_ref, wh_ref, bcat_ref, wp_ref, lng_ref, lnb_ref,
             w1_ref, we1_ref,
             hid_ref, cel_ref, q_ref, qe2_ref, kv_ref, r_ref):
    a = x_ref[:] + mx_ref[:]
    bh = h_ref[:] + mh_ref[:]
    c = c_ref[:]
    z = (jnp.dot(a, wx_ref[:], preferred_element_type=jnp.float32)
         + jnp.dot(bh, wh_ref[:], preferred_element_type=jnp.float32)
         + bcat_ref[:])
    zi = z[:, 0:D]
    zf = z[:, D:2 * D]
    zg = z[:, 2 * D:3 * D]
    zo = z[:, 3 * D:4 * D]
    wpi = wp_ref[0:1, :]
    wpf = wp_ref[1:2, :]
    wpo = wp_ref[2:3, :]
    ig = jax.nn.sigmoid(zi + wpi * c)
    fg = jax.nn.sigmoid(zf + wpf * c)
    gg = jnp.tanh(zg)
    cn = fg * c + ig * gg
    og = jax.nn.sigmoid(zo + wpo * cn)
    hn = og * jnp.tanh(cn)
    hid_ref[:] = _ln_rows(hn, lng_ref[1:2, :], lnb_ref[1:2, :])
    cel_ref[:] = _ln_rows(cn, lng_ref[2:3, :], lnb_ref[2:3, :])
    o0 = jax.nn.relu(_ln_rows(hn, lng_ref[0:1, :], lnb_ref[0:1, :]))
    x130 = jnp.concatenate([o0, sk_ref[:]], axis=1)
    qkvr = jnp.dot(x130, w1_ref[:], preferred_element_type=jnp.float32)
    inv = np.float32(1.0 / np.sqrt(float(D)))
    q = qkvr[:, 0:D] * inv
    r_ref[:] = qkvr[:, 3 * D:4 * D]
    q_ref[:] = q
    qe2_ref[:] = jnp.dot(q, we1_ref[:].T, preferred_element_type=jnp.float32)
    kv_ref[:] = qkvr[:, D:3 * D]


def _k2_call(xp, mx, hp, mh, cp, skp, wx, wh, bcat, wp, lng, lnb, w1, we1):
    BLK = 512
    grid = (NP // BLK,)
    row = lambda i: (i, 0)
    full = lambda i: (0, 0)
    return pl.pallas_call(
        _k2_body,
        grid=grid,
        in_specs=[
            pl.BlockSpec((BLK, D), row), pl.BlockSpec((BLK, D), row),
            pl.BlockSpec((BLK, D), row), pl.BlockSpec((BLK, D), row),
            pl.BlockSpec((BLK, D), row), pl.BlockSpec((BLK, SK), row),
            pl.BlockSpec((D, 4 * D), full), pl.BlockSpec((D, 4 * D), full),
            pl.BlockSpec((1, 4 * D), full), pl.BlockSpec((3, D), full),
            pl.BlockSpec((3, D), full), pl.BlockSpec((3, D), full),
            pl.BlockSpec((D + SK, 4 * D), full), pl.BlockSpec((2, D), full),
        ],
        out_specs=[
            pl.BlockSpec((BLK, D), row), pl.BlockSpec((BLK, D), row),
            pl.BlockSpec((BLK, D), row), pl.BlockSpec((BLK, 2), row),
            pl.BlockSpec((BLK, KVW), row), pl.BlockSpec((BLK, D), row),
        ],
        out_shape=[
            jax.ShapeDtypeStruct((NP, D), jnp.float32),
            jax.ShapeDtypeStruct((NP, D), jnp.float32),
            jax.ShapeDtypeStruct((NP, D), jnp.float32),
            jax.ShapeDtypeStruct((NP, 2), jnp.float32),
            jax.ShapeDtypeStruct((NP, KVW), jnp.float32),
            jax.ShapeDtypeStruct((NP, D), jnp.float32),
        ],
    )(xp, mx, hp, mh, cp, skp, wx, wh, bcat, wp, lng, lnb, w1, we1)


def _k4_body(acc_ref, dst_ref, r_ref, we1_ref, b1_ref, w2_ref, qkvr2_ref):
    numv = acc_ref[:]
    st = dst_ref[:]
    den = st[:, 0:1]
    dew = st[:, 1:3]
    agg = (numv + jnp.dot(dew, we1_ref[:], preferred_element_type=jnp.float32)) \
        / (den + 1e-16)
    out1 = jax.nn.relu(agg + r_ref[:] + b1_ref[:])
    qkvr2_ref[:] = jnp.dot(out1, w2_ref[:], preferred_element_type=jnp.float32)


def _k4_call(acc, dstats_t, r1, we1, b1, w2):
    BLK = 512
    grid = (NP // BLK,)
    return pl.pallas_call(
        _k4_body,
        grid=grid,
        in_specs=[
            pl.BlockSpec((BLK, D), lambda i: (i, 0)),
            pl.BlockSpec((BLK, 3), lambda i: (i, 0)),
            pl.BlockSpec((BLK, D), lambda i: (i, 0)),
            pl.BlockSpec((2, D), lambda i: (0, 0)),
            pl.BlockSpec((1, D), lambda i: (0, 0)),
            pl.BlockSpec((D, 4), lambda i: (0, 0)),
        ],
        out_specs=pl.BlockSpec((BLK, 4), lambda i: (i, 0)),
        out_shape=jax.ShapeDtypeStruct((NP, 4), jnp.float32),
    )(acc, dstats_t, r1, we1, b1, w2)


def _k6_body(dn_ref, r2_ref, b2_ref, out_ref):
    den = dn_ref[0:1, :]
    num = dn_ref[1:2, :]
    for t in range(1, 32):
        den = den + dn_ref[2 * t:2 * t + 1, :]
        num = num + dn_ref[2 * t + 1:2 * t + 2, :]
    out_ref[:] = jax.nn.sigmoid(num / (den + 1e-16) + r2_ref[:] + b2_ref[0, 0])


def _k6_call(dn, r2, b2):
    return pl.pallas_call(
        _k6_body,
        in_specs=[
            pl.BlockSpec((64, NP), lambda: (0, 0)),
            pl.BlockSpec((1, NP), lambda: (0, 0)),
            pl.BlockSpec((1, 1), lambda: (0, 0)),
        ],
        out_specs=pl.BlockSpec((1, NP), lambda: (0, 0)),
        out_shape=jax.ShapeDtypeStruct((1, NP), jnp.float32),
        grid=(),
    )(dn, r2, b2)


@jax.jit
def _run(X, edge_index, edge_weight, skip, H, C, p):
    src = edge_index[0].astype(jnp.int32)
    dst = edge_index[1].astype(jnp.int32)
    ew0 = edge_weight[:, 0]

    padn = lambda x: jnp.pad(x, ((0, NP - N), (0, 0)))
    xp = padn(X)
    hp = padn(H[0])
    cp = padn(C[0])
    skp = padn(skip)
    table = jnp.concatenate([xp, hp], axis=0)

    msg = _k1_msg(table, src, dst, ew0)

    wx = jnp.concatenate([p['Wx_i'], p['Wx_f'], p['Wx_c'], p['Wx_o']], axis=1)
    wh = jnp.concatenate([p['Wh_i'], p['Wh_f'], p['Wh_c'], p['Wh_o']], axis=1)
    bcat = jnp.concatenate([p['b_i'], p['b_f'], p['b_c'], p['b_o']])[None]
    wp = jnp.stack([p['wp_i'], p['wp_f'], p['wp_o']])
    lng = jnp.stack([p['ln_g_o'], p['ln_g_h'], p['ln_g_c']])
    lnb = jnp.stack([p['ln_b_o'], p['ln_b_h'], p['ln_b_c']])
    w1 = jnp.concatenate([p['Wq1'], p['Wk1'], p['Wv1'], p['Wr1']], axis=1)

    hid, cel, q_t, qe2, kv_t, r1 = _k2_call(
        xp, msg[0], hp, msg[1], cp, skp, wx, wh, bcat, wp, lng, lnb,
        w1, p['We1'])

    w2vec = jnp.pad(p['We2'][:, 0], (0, 14))
    acc, dn3, _part, e2 = _k3_tconv1(q_t, kv_t, qe2[:, 0], qe2[:, 1],
                                     src, dst, ew0, edge_weight[:, 1], w2vec)

    w2cat = jnp.concatenate([p['Wq2'], p['Wk2'], p['Wv2'], p['Wr2']], axis=1)
    qkvr2 = _k4_call(acc, dn3.reshape(3, NP).T, r1, p['We1'], p['b1'][None],
                     w2cat)

    dn = _k5_tconv2(qkvr2[:, 0], qkvr2[:, 1], qkvr2[:, 2], src, dst, e2)

    out = _k6_call(dn.reshape(64, NP), qkvr2[:, 3][None], p['b2'][None])
    return (out[0, :N, None], hid[:N][None], cel[:N][None])


def kernel(X, edge_index, edge_weight, skip, H, C, params):
    return _run(X, edge_index, edge_weight, skip, H, C, params)
